# Initial kernel scaffold; baseline (speedup 1.0000x reference)
import jax
import jax.numpy as jnp
from jax.experimental import pallas as pl


def kernel(x, grid_0_0, grid_0_1, grid_0_2, grid_1_0, grid_1_1, grid_1_2, grid_2_0, grid_2_1, grid_2_2, grid_3_0, grid_3_1, grid_3_2, W1, W2, W3):
    raise NotImplementedError("write your pallas kernel here")


# speedup vs baseline: 58.2614x; 58.2614x over previous
import functools

import jax
import jax.numpy as jnp
from jax import lax
from jax.experimental import pallas as pl
from jax.experimental.pallas import tpu as pltpu
from jax.experimental.pallas import tpu_sc as plsc

N_PTS = 262144
C = 32
RES = (64, 128, 256, 512)
COMBS = ((0, 1), (0, 2), (1, 2))
L = 16
NW = 32
PTS_PER_W = N_PTS // NW
B = 128
NBLK = PTS_PER_W // B

_sc_mesh = plsc.VectorSubcoreMesh(core_axis_name="c", subcore_axis_name="s",
                                  num_cores=2, num_subcores=16)


@functools.partial(
    pl.kernel,
    out_type=jax.ShapeDtypeStruct((N_PTS, 4 * C), jnp.float32),
    mesh=_sc_mesh,
    scratch_types=[
        pltpu.VMEM((3, B), jnp.float32),
        pltpu.VMEM((3, 4, B), jnp.int32),
        pltpu.VMEM((3, 2, B + L), jnp.float32),
        pltpu.VMEM((3, 4, B, C), jnp.float32),
        pltpu.VMEM((B, 4 * C), jnp.float32),
        pltpu.SemaphoreType.DMA,
    ],
    compiler_params=pltpu.CompilerParams(use_tc_tiling_on_sc=False),
)
def _sc_encode(x0h, x1h, x2h,
               t00, t01, t02, t10, t11, t12, t20, t21, t22, t30, t31, t32,
               featsh, xv, idxv, wv, gb, fb, sem):
    tables = (t00, t01, t02, t10, t11, t12, t20, t21, t22, t30, t31, t32)
    wid = lax.axis_index("s") * 2 + lax.axis_index("c")
    base0 = wid * PTS_PER_W

    def block_body(blk, carry):
        base = base0 + blk * B
        pltpu.sync_copy(x0h.at[pl.ds(base, B)], xv.at[0])
        pltpu.sync_copy(x1h.at[pl.ds(base, B)], xv.at[1])
        pltpu.sync_copy(x2h.at[pl.ds(base, B)], xv.at[2])
        for s in range(4):
            R = RES[s]

            def idx_body(g, c, R=R):
                sl = pl.ds(g * L, L)
                iis = []
                ws = []
                for d in range(3):
                    f = (xv[d, sl] + 1.0) * (0.5 * (R - 1))
                    i = jnp.minimum(f.astype(jnp.int32), R - 2)
                    iis.append(i)
                    ws.append(f - i.astype(jnp.float32))
                for p, (a, b) in enumerate(COMBS):
                    bi = iis[b] * R + iis[a]
                    idxv[p, 0, sl] = bi
                    idxv[p, 1, sl] = bi + 1
                    idxv[p, 2, sl] = bi + R
                    idxv[p, 3, sl] = bi + (R + 1)
                    wv[p, 0, sl] = ws[a]
                    wv[p, 1, sl] = ws[b]
                return c

            lax.fori_loop(0, B // L, idx_body, 0)

            copies = []
            for p in range(3):
                for k in range(4):
                    copies.append(pltpu.async_copy(
                        tables[s * 3 + p].at[idxv.at[p, k]], gb.at[p, k], sem))
            for cp in copies:
                cp.wait()

            def comb_body(g, c, s=s):
                gsl = pl.ds(g * L, L)
                wvecs = [(wv[p, 0, gsl], wv[p, 1, gsl]) for p in range(3)]
                for jj in range(L):
                    j = g * L + jj
                    for h in range(2):
                        hsl = pl.ds(h * L, L)
                        f = None
                        for p in range(3):
                            wx = wvecs[p][0][jj]
                            wy = wvecs[p][1][jj]
                            g00 = gb[p, 0, j, hsl]
                            g01 = gb[p, 1, j, hsl]
                            g10 = gb[p, 2, j, hsl]
                            g11 = gb[p, 3, j, hsl]
                            gx0 = g00 + (g01 - g00) * wx
                            gx1 = g10 + (g11 - g10) * wx
                            v = gx0 + (gx1 - gx0) * wy
                            f = v if f is None else f * v
                        fb[j, pl.ds(s * C + h * L, L)] = f
                return c

            lax.fori_loop(0, B // L, comb_body, 0)
        pltpu.sync_copy(fb, featsh.at[pl.ds(base, B)])
        return carry

    lax.fori_loop(0, NBLK, block_body, 0)


BN = 2048


def _mlp_body(f_ref, w1_ref, w2_ref, w3_ref, o_ref):
    h = jnp.maximum(jnp.dot(f_ref[...], w1_ref[...],
                            preferred_element_type=jnp.float32), 0.0)
    h = jnp.maximum(jnp.dot(h, w2_ref[...],
                            preferred_element_type=jnp.float32), 0.0)
    o_ref[...] = jnp.dot(h, w3_ref[...], preferred_element_type=jnp.float32)


_mlp = pl.pallas_call(
    _mlp_body,
    grid=(N_PTS // BN,),
    in_specs=[
        pl.BlockSpec((BN, 4 * C), lambda i: (i, 0)),
        pl.BlockSpec((4 * C, 64), lambda i: (0, 0)),
        pl.BlockSpec((64, 64), lambda i: (0, 0)),
        pl.BlockSpec((64, 1), lambda i: (0, 0)),
    ],
    out_specs=pl.BlockSpec((BN, 1), lambda i: (i, 0)),
    out_shape=jax.ShapeDtypeStruct((N_PTS, 1), jnp.float32),
)


def kernel(x, grid_0_0, grid_0_1, grid_0_2, grid_1_0, grid_1_1, grid_1_2,
           grid_2_0, grid_2_1, grid_2_2, grid_3_0, grid_3_1, grid_3_2,
           W1, W2, W3):
    grids = (grid_0_0, grid_0_1, grid_0_2, grid_1_0, grid_1_1, grid_1_2,
             grid_2_0, grid_2_1, grid_2_2, grid_3_0, grid_3_1, grid_3_2)
    tables = [g.transpose(1, 2, 0).reshape(-1, C) for g in grids]
    x0 = x[:, 0]
    x1 = x[:, 1]
    x2 = x[:, 2]
    feats = _sc_encode(x0, x1, x2, *tables)
    return _mlp(feats, W1, W2, W3)


# speedup vs baseline: 68.8596x; 1.1819x over previous
import functools

import jax
import jax.numpy as jnp
from jax import lax
from jax.experimental import pallas as pl
from jax.experimental.pallas import tpu as pltpu
from jax.experimental.pallas import tpu_sc as plsc

N_PTS = 262144
C = 32
RES = (64, 128, 256, 512)
COMBS = ((0, 1), (0, 2), (1, 2))
L = 16
NW = 32
PTS_PER_W = N_PTS // NW
B = 128
NBLK = PTS_PER_W // B

_sc_mesh = plsc.VectorSubcoreMesh(core_axis_name="c", subcore_axis_name="s",
                                  num_cores=2, num_subcores=16)


@functools.partial(
    pl.kernel,
    out_type=jax.ShapeDtypeStruct((N_PTS, 4 * C), jnp.float32),
    mesh=_sc_mesh,
    scratch_types=[
        pltpu.VMEM((3, B), jnp.float32),
        pltpu.VMEM((2, 3, 4, B), jnp.int32),
        pltpu.VMEM((2, 3, 2, B + L), jnp.float32),
        pltpu.VMEM((2, 3, 4, B, C // 2), jnp.int32),
        pltpu.VMEM((B, 4 * C), jnp.float32),
        pltpu.SemaphoreType.DMA,
        pltpu.SemaphoreType.DMA,
    ],
    compiler_params=pltpu.CompilerParams(use_tc_tiling_on_sc=False),
)
def _sc_encode(x0h, x1h, x2h,
               t00, t01, t02, t10, t11, t12, t20, t21, t22, t30, t31, t32,
               featsh, xv, idxv, wv, gb, fb, sem0, sem1):
    tables = (t00, t01, t02, t10, t11, t12, t20, t21, t22, t30, t31, t32)
    sems = (sem0, sem1)
    wid = lax.axis_index("s") * 2 + lax.axis_index("c")
    base0 = wid * PTS_PER_W

    def load_x(base):
        pltpu.sync_copy(x0h.at[pl.ds(base, B)], xv.at[0])
        pltpu.sync_copy(x1h.at[pl.ds(base, B)], xv.at[1])
        pltpu.sync_copy(x2h.at[pl.ds(base, B)], xv.at[2])

    def compute_idx(s, slot):
        R = RES[s]

        def idx_body(g, c):
            sl = pl.ds(g * L, L)
            iis = []
            ws = []
            for d in range(3):
                f = (xv[d, sl] + 1.0) * (0.5 * (R - 1))
                i = jnp.minimum(f.astype(jnp.int32), R - 2)
                iis.append(i)
                ws.append(f - i.astype(jnp.float32))
            for p, (a, b) in enumerate(COMBS):
                bi = iis[b] * R + iis[a]
                idxv[slot, p, 0, sl] = bi
                idxv[slot, p, 1, sl] = bi + 1
                idxv[slot, p, 2, sl] = bi + R
                idxv[slot, p, 3, sl] = bi + (R + 1)
                wv[slot, p, 0, sl] = ws[a]
                wv[slot, p, 1, sl] = ws[b]
            return c

        lax.fori_loop(0, B // L, idx_body, 0)

    def fire(s, slot):
        for p in range(3):
            for k in range(4):
                pltpu.async_copy(tables[s * 3 + p].at[idxv.at[slot, p, k]],
                                 gb.at[slot, p, k], sems[slot])

    def drain(slot):
        for _ in range(12):
            pltpu.make_async_copy(tables[0].at[pl.ds(0, B)],
                                  gb.at[slot, 0, 0], sems[slot]).wait()

    def combine(s, slot):
        def comb_body(g, c):
            gsl = pl.ds(g * L, L)
            wvecs = [(wv[slot, p, 0, gsl], wv[slot, p, 1, gsl])
                     for p in range(3)]
            for jj in range(L):
                j = g * L + jj
                fe = None
                fo = None
                for p in range(3):
                    wx = wvecs[p][0][jj]
                    wy = wvecs[p][1][jj]
                    un = []
                    for k in range(4):
                        w32 = gb[slot, p, k, j, :]
                        ev = jax.lax.bitcast_convert_type(
                            w32 << 16, jnp.float32)
                        od = jax.lax.bitcast_convert_type(
                            w32 & jnp.int32(-65536), jnp.float32)
                        un.append((ev, od))
                    for h in range(2):
                        g00 = un[0][h]
                        g01 = un[1][h]
                        g10 = un[2][h]
                        g11 = un[3][h]
                        gx0 = g00 + (g01 - g00) * wx
                        gx1 = g10 + (g11 - g10) * wx
                        v = gx0 + (gx1 - gx0) * wy
                        if h == 0:
                            fe = v if fe is None else fe * v
                        else:
                            fo = v if fo is None else fo * v
                fb[j, pl.ds(s * C, L)] = fe
                fb[j, pl.ds(s * C + L, L)] = fo
            return c

        lax.fori_loop(0, B // L, comb_body, 0)

    load_x(base0)
    compute_idx(0, 0)
    fire(0, 0)

    def block_body(blk, carry):
        base = base0 + blk * B
        for s in range(4):
            if s < 3:
                compute_idx(s + 1, (s + 1) % 2)
                fire(s + 1, (s + 1) % 2)
            else:
                @pl.when(blk < NBLK - 1)
                def _prefire():
                    load_x(base + B)
                    compute_idx(0, 0)
                    fire(0, 0)
            drain(s % 2)
            combine(s, s % 2)
        pltpu.sync_copy(fb, featsh.at[pl.ds(base, B)])
        return carry

    lax.fori_loop(0, NBLK, block_body, 0)


BN = 2048


def _mlp_body(f_ref, w1_ref, w2_ref, w3_ref, o_ref):
    h = jnp.maximum(jnp.dot(f_ref[...], w1_ref[...],
                            preferred_element_type=jnp.float32), 0.0)
    h = jnp.maximum(jnp.dot(h, w2_ref[...],
                            preferred_element_type=jnp.float32), 0.0)
    o_ref[...] = jnp.dot(h, w3_ref[...], preferred_element_type=jnp.float32)


_mlp = pl.pallas_call(
    _mlp_body,
    grid=(N_PTS // BN,),
    in_specs=[
        pl.BlockSpec((BN, 4 * C), lambda i: (i, 0)),
        pl.BlockSpec((4 * C, 64), lambda i: (0, 0)),
        pl.BlockSpec((64, 64), lambda i: (0, 0)),
        pl.BlockSpec((64, 1), lambda i: (0, 0)),
    ],
    out_specs=pl.BlockSpec((BN, 1), lambda i: (i, 0)),
    out_shape=jax.ShapeDtypeStruct((N_PTS, 1), jnp.float32),
)

_PERM = [s * C + k for s in range(4)
         for k in list(range(0, C, 2)) + list(range(1, C, 2))]


def kernel(x, grid_0_0, grid_0_1, grid_0_2, grid_1_0, grid_1_1, grid_1_2,
           grid_2_0, grid_2_1, grid_2_2, grid_3_0, grid_3_1, grid_3_2,
           W1, W2, W3):
    grids = (grid_0_0, grid_0_1, grid_0_2, grid_1_0, grid_1_1, grid_1_2,
             grid_2_0, grid_2_1, grid_2_2, grid_3_0, grid_3_1, grid_3_2)
    tables = [
        jax.lax.bitcast_convert_type(
            g.transpose(1, 2, 0).reshape(-1, C // 2, 2).astype(jnp.bfloat16),
            jnp.int32)
        for g in grids]
    x0 = x[:, 0]
    x1 = x[:, 1]
    x2 = x[:, 2]
    feats = _sc_encode(x0, x1, x2, *tables)
    w1p = W1[jnp.array(_PERM), :]
    return _mlp(feats, w1p, W2, W3)
